# Initial kernel scaffold; baseline (speedup 1.0000x reference)
import jax
import jax.numpy as jnp
from jax.experimental import pallas as pl


def kernel(x, edge_index, W1, b1):
    raise NotImplementedError("write your pallas kernel here")


# speedup vs baseline: 5.3627x; 5.3627x over previous
import functools

import jax
import jax.numpy as jnp
from jax import lax
from jax.experimental import pallas as pl
from jax.experimental.pallas import tpu as pltpu
from jax.experimental.pallas import tpu_sc as plsc

N = 10000
NP = 10240
D = 128
E = 320000
K_HOPS = 10
ALPHA = 0.15
SCALE = 1.8
BETA = 1.0 - ALPHA

NC, NS = 2, 16
NTILES = NC * NS
EPT = E // NTILES
HALF = NP // NC
CAP = 10240
NCH = CAP // 128
TR = 1024
RPT = HALF // NS
AGR = HALF + TR
ZPT = AGR // NS

_mesh = plsc.VectorSubcoreMesh(core_axis_name="c", subcore_axis_name="s")


@functools.partial(
    pl.kernel,
    out_type=(
        jax.ShapeDtypeStruct((NC, NP), jnp.float32),
        jax.ShapeDtypeStruct((NTILES, NCH, 128), jnp.int32),
        jax.ShapeDtypeStruct((NTILES, NC, NCH, 128), jnp.int32),
    ),
    mesh=_mesh,
    scratch_types=(
        pltpu.VMEM_SHARED((NP,), jnp.float32),
        pltpu.VMEM((CAP,), jnp.int32),
        pltpu.VMEM((CAP,), jnp.int32),
        pltpu.VMEM((NCH, 128), jnp.int32),
        pltpu.VMEM((NCH, 128), jnp.float32),
        pltpu.VMEM((NCH, 128), jnp.int32),
        pltpu.VMEM((NP // NS,), jnp.float32),
    ),
)
def _prep(erow, ecol, degp_o, lrow_o, lcol_o,
          deg_sh, rows1, cols1, idx2, upd2, buf2d, zb):
    c = lax.axis_index("c")
    s = lax.axis_index("s")
    w = s * NC + c
    i16 = lax.iota(jnp.int32, 16)
    zeros16 = jnp.zeros((16,), jnp.float32)
    ones16 = jnp.ones((16,), jnp.float32)

    pltpu.sync_copy(erow.at[pl.ds(w * EPT, EPT)], rows1.at[pl.ds(0, EPT)])
    pltpu.sync_copy(ecol.at[pl.ds(w * EPT, EPT)], cols1.at[pl.ds(0, EPT)])

    def _zb(i, _):
        zb[pl.ds(i * 16, 16)] = zeros16
        return 0
    lax.fori_loop(0, (NP // NS) // 16, _zb, 0)
    pltpu.sync_copy(zb, deg_sh.at[pl.ds(s * (NP // NS), NP // NS)])

    for u in range((CAP - EPT) // 16):
        rows1[pl.ds(EPT + u * 16, 16)] = N + (i16 + u * 16 + w * 16) % (NP - N)
        cols1[pl.ds(EPT + u * 16, 16)] = (i16 * 8 + u * 128) % NP

    def _fill(t, _):
        idx2.at[t // 8][pl.ds((t % 8) * 16, 16)] = cols1[pl.ds(t * 16, 16)]
        upd2.at[t // 8][pl.ds((t % 8) * 16, 16)] = ones16
        return 0
    lax.fori_loop(0, EPT // 16, _fill, 0)

    def _fillz(t, _):
        idx2.at[t // 8][pl.ds((t % 8) * 16, 16)] = cols1[pl.ds(t * 16, 16)]
        upd2.at[t // 8][pl.ds((t % 8) * 16, 16)] = zeros16
        return 0
    lax.fori_loop(EPT // 16, CAP // 16, _fillz, 0)

    plsc.subcore_barrier()

    def _deg(j, _):
        pltpu.sync_copy(upd2.at[j], deg_sh.at[idx2.at[j]], add=True)
        return 0
    lax.fori_loop(0, NCH, _deg, 0)

    plsc.subcore_barrier()
    pltpu.sync_copy(deg_sh.at[pl.ds(s * (NP // NS), NP // NS)], zb)
    pltpu.sync_copy(zb, degp_o.at[c, pl.ds(s * (NP // NS), NP // NS)])

    def _rw(t, _):
        buf2d.at[t // 8][pl.ds((t % 8) * 16, 16)] = rows1[pl.ds(t * 16, 16)]
        return 0
    lax.fori_loop(0, CAP // 16, _rw, 0)
    pltpu.sync_copy(buf2d, lrow_o.at[w])

    for u in range((CAP - EPT) // 16):
        cols1[pl.ds(EPT + u * 16, 16)] = jnp.full((16,), 2 * NP, jnp.int32)

    def _rm0(t, _):
        col16 = cols1[pl.ds(t * 16, 16)]
        trash = HALF + (i16 * 64 + t) % TR
        m = col16 < HALF
        buf2d.at[t // 8][pl.ds((t % 8) * 16, 16)] = jnp.where(m, col16, trash)
        return 0
    lax.fori_loop(0, CAP // 16, _rm0, 0)
    pltpu.sync_copy(buf2d, lcol_o.at[w, 0])

    def _rm1(t, _):
        col16 = cols1[pl.ds(t * 16, 16)]
        trash = HALF + (i16 * 64 + t) % TR
        m = (col16 >= HALF) & (col16 < NP)
        buf2d.at[t // 8][pl.ds((t % 8) * 16, 16)] = jnp.where(
            m, col16 - HALF, trash)
        return 0
    lax.fori_loop(0, CAP // 16, _rm1, 0)
    pltpu.sync_copy(buf2d, lcol_o.at[w, 1])


_BLK = 256


def _tc_body(x_ref, w_ref, b_ref, degp_ref,
             outs_ref, aggi_ref, hh_ref, dinvr_ref, selfcr_ref):
    i = pl.program_id(0)
    h = jnp.dot(x_ref[...], w_ref[...], preferred_element_type=jnp.float32)
    h = h + b_ref[...][None, :]
    nrm2 = jnp.sum(h * h, axis=1, keepdims=True)
    h = h * (SCALE * lax.rsqrt(jnp.maximum(nrm2, 1e-24)))
    rows = i * _BLK + lax.broadcasted_iota(jnp.int32, (_BLK, 1), 0)
    mask = (rows < N).astype(jnp.float32)
    h = h * mask
    deg = degp_ref[0, :] + degp_ref[1, :] + 1.0
    dinv = lax.rsqrt(deg) * mask[:, 0]
    selfc = BETA * dinv * dinv
    hh = ALPHA * h
    outs_ref[...] = h * dinv[:, None]
    hh_ref[...] = hh
    aggi_ref[...] = selfc[:, None] * h + hh
    dinvr_ref[...] = jnp.broadcast_to(dinv[:, None], (_BLK, 16))
    selfcr_ref[...] = jnp.broadcast_to(selfc[:, None], (_BLK, 16))


_tc_prep = pl.pallas_call(
    _tc_body,
    grid=(NP // _BLK,),
    in_specs=[
        pl.BlockSpec((_BLK, D), lambda i: (i, 0)),
        pl.BlockSpec((D, D), lambda i: (0, 0)),
        pl.BlockSpec((D,), lambda i: (0,)),
        pl.BlockSpec((NC, _BLK), lambda i: (0, i)),
    ],
    out_specs=[
        pl.BlockSpec((_BLK, D), lambda i: (i, 0)),
        pl.BlockSpec((_BLK, D), lambda i: (i, 0)),
        pl.BlockSpec((_BLK, D), lambda i: (i, 0)),
        pl.BlockSpec((_BLK, 16), lambda i: (i, 0)),
        pl.BlockSpec((_BLK, 16), lambda i: (i, 0)),
    ],
    out_shape=[
        jax.ShapeDtypeStruct((NP, D), jnp.float32),
        jax.ShapeDtypeStruct((NP, D), jnp.float32),
        jax.ShapeDtypeStruct((NP, D), jnp.float32),
        jax.ShapeDtypeStruct((NP, 16), jnp.float32),
        jax.ShapeDtypeStruct((NP, 16), jnp.float32),
    ],
)


@functools.partial(
    pl.kernel,
    out_type=(
        jax.ShapeDtypeStruct((NP, D), jnp.float32),
        jax.ShapeDtypeStruct((NP, D), jnp.float32),
        jax.ShapeDtypeStruct((NP, D), jnp.float32),
    ),
    mesh=_mesh,
    scratch_types=(
        pltpu.VMEM_SHARED((AGR, D), jnp.float32),
        pltpu.VMEM((128,), jnp.int32),
        pltpu.VMEM((128,), jnp.int32),
        pltpu.VMEM((128, D), jnp.float32),
        pltpu.VMEM((128, D), jnp.float32),
        pltpu.VMEM((64, 16), jnp.float32),
        pltpu.VMEM((64, 16), jnp.float32),
        pltpu.SemaphoreType.DMA,
        pltpu.SemaphoreType.DMA,
    ),
)
def _hop(outs_i, aggi_i, hh_i, dinvr_i, selfcr_i, lrow_i, lcol_i,
         out_o, outs_o, aggi_o,
         agg_sh, lrv, lcv, bA, bB, dv_v, sc_v,
         semg, sems):
    c = lax.axis_index("c")
    s = lax.axis_index("s")
    zeros16 = jnp.zeros((16,), jnp.float32)

    def _zb(t, _):
        bA.at[t // 8][pl.ds((t % 8) * 16, 16)] = zeros16
        return 0
    lax.fori_loop(0, (128 * D) // 16, _zb, 0)
    pltpu.sync_copy(bA, agg_sh.at[pl.ds(s * ZPT, 128)])
    pltpu.sync_copy(bA, agg_sh.at[pl.ds(s * ZPT + 128, 128)])
    pltpu.sync_copy(bA, agg_sh.at[pl.ds(s * ZPT + 256, 128)])
    plsc.subcore_barrier()

    def _slot(w):
        def body(k, _):
            pltpu.sync_copy(lrow_i.at[w, k], lrv)
            pltpu.sync_copy(lcol_i.at[w, c, k], lcv)
            pltpu.async_copy(outs_i.at[lrv], bA, semg).wait()
            pltpu.async_copy(bA, agg_sh.at[lcv], sems, add=True).wait()
            return 0
        lax.fori_loop(0, NCH, body, 0)
    _slot(2 * s)
    _slot(2 * s + 1)
    plsc.subcore_barrier()

    for j in range(RPT // 64):
        gbase = c * HALF + s * RPT + j * 64
        lbase = s * RPT + j * 64
        pltpu.sync_copy(agg_sh.at[pl.ds(lbase, 64)], bA.at[pl.ds(0, 64)])
        pltpu.sync_copy(aggi_i.at[pl.ds(gbase, 64)], bA.at[pl.ds(64, 64)])
        pltpu.sync_copy(hh_i.at[pl.ds(gbase, 64)], bB.at[pl.ds(0, 64)])
        pltpu.sync_copy(dinvr_i.at[pl.ds(gbase, 64)], dv_v)
        pltpu.sync_copy(selfcr_i.at[pl.ds(gbase, 64)], sc_v)

        def _row(r, _):
            dv = dv_v.at[r][pl.ds(0, 16)]
            sc_ = sc_v.at[r][pl.ds(0, 16)]
            for q in range(D // 16):
                a = bA.at[r][pl.ds(q * 16, 16)]
                ai = bA.at[64 + r][pl.ds(q * 16, 16)]
                hhv = bB.at[r][pl.ds(q * 16, 16)]
                on = (BETA * dv) * a + ai
                bB.at[64 + r][pl.ds(q * 16, 16)] = on
                bA.at[r][pl.ds(q * 16, 16)] = on * dv
                bA.at[64 + r][pl.ds(q * 16, 16)] = on * sc_ + hhv
            return 0
        lax.fori_loop(0, 64, _row, 0)

        pltpu.sync_copy(bB.at[pl.ds(64, 64)], out_o.at[pl.ds(gbase, 64)])
        pltpu.sync_copy(bA.at[pl.ds(0, 64)], outs_o.at[pl.ds(gbase, 64)])
        pltpu.sync_copy(bA.at[pl.ds(64, 64)], aggi_o.at[pl.ds(gbase, 64)])


def kernel(x, edge_index, W1, b1):
    xp = jnp.pad(x, ((0, NP - N), (0, 0)))
    degp, lrow, lcol = _prep(edge_index[0], edge_index[1])
    outs, aggi, hh, dinvr, selfcr = _tc_prep(xp, W1, b1, degp)
    out = None
    for _ in range(K_HOPS):
        out, outs, aggi = _hop(outs, aggi, hh, dinvr, selfcr, lrow, lcol)
    return out[:N]


# speedup vs baseline: 8.4844x; 1.5821x over previous
import functools

import jax
import jax.numpy as jnp
from jax import lax
from jax.experimental import pallas as pl
from jax.experimental.pallas import tpu as pltpu
from jax.experimental.pallas import tpu_sc as plsc

N = 10000
NP = 10240
D = 128
E = 320000
K_HOPS = 10
ALPHA = 0.15
SCALE = 1.8
BETA = 1.0 - ALPHA

NC, NS = 2, 16
NTILES = NC * NS
EPT = E // NTILES
HALF = NP // NC
CAP = 10240
NCH = CAP // 128
TR = 1024
RPT = HALF // NS
AGR = HALF + TR
ZPT = AGR // NS

_mesh = plsc.VectorSubcoreMesh(core_axis_name="c", subcore_axis_name="s")


@functools.partial(
    pl.kernel,
    out_type=(
        jax.ShapeDtypeStruct((NC, NP), jnp.float32),
        jax.ShapeDtypeStruct((NTILES, NCH, 128), jnp.int32),
        jax.ShapeDtypeStruct((NTILES, NC, NCH, 128), jnp.int32),
    ),
    mesh=_mesh,
    scratch_types=(
        pltpu.VMEM_SHARED((NP,), jnp.float32),
        pltpu.VMEM((CAP,), jnp.int32),
        pltpu.VMEM((CAP,), jnp.int32),
        pltpu.VMEM((NCH, 128), jnp.int32),
        pltpu.VMEM((NCH, 128), jnp.float32),
        pltpu.VMEM((NCH, 128), jnp.int32),
        pltpu.VMEM((NP // NS,), jnp.float32),
    ),
)
def _prep(erow, ecol, degp_o, lrow_o, lcol_o,
          deg_sh, rows1, cols1, idx2, upd2, buf2d, zb):
    c = lax.axis_index("c")
    s = lax.axis_index("s")
    w = s * NC + c
    i16 = lax.iota(jnp.int32, 16)
    zeros16 = jnp.zeros((16,), jnp.float32)
    ones16 = jnp.ones((16,), jnp.float32)

    pltpu.sync_copy(erow.at[pl.ds(w * EPT, EPT)], rows1.at[pl.ds(0, EPT)])
    pltpu.sync_copy(ecol.at[pl.ds(w * EPT, EPT)], cols1.at[pl.ds(0, EPT)])

    def _zb(i, _):
        zb[pl.ds(i * 16, 16)] = zeros16
        return 0
    lax.fori_loop(0, (NP // NS) // 16, _zb, 0)
    pltpu.sync_copy(zb, deg_sh.at[pl.ds(s * (NP // NS), NP // NS)])

    for u in range((CAP - EPT) // 16):
        rows1[pl.ds(EPT + u * 16, 16)] = N + (i16 + u * 16 + w * 16) % (NP - N)
        cols1[pl.ds(EPT + u * 16, 16)] = (i16 * 8 + u * 128) % NP

    def _fill(t, _):
        idx2.at[t // 8][pl.ds((t % 8) * 16, 16)] = cols1[pl.ds(t * 16, 16)]
        upd2.at[t // 8][pl.ds((t % 8) * 16, 16)] = ones16
        return 0
    lax.fori_loop(0, EPT // 16, _fill, 0)

    def _fillz(t, _):
        idx2.at[t // 8][pl.ds((t % 8) * 16, 16)] = cols1[pl.ds(t * 16, 16)]
        upd2.at[t // 8][pl.ds((t % 8) * 16, 16)] = zeros16
        return 0
    lax.fori_loop(EPT // 16, CAP // 16, _fillz, 0)

    plsc.subcore_barrier()

    def _deg(j, _):
        pltpu.sync_copy(upd2.at[j], deg_sh.at[idx2.at[j]], add=True)
        return 0
    lax.fori_loop(0, NCH, _deg, 0)

    plsc.subcore_barrier()
    pltpu.sync_copy(deg_sh.at[pl.ds(s * (NP // NS), NP // NS)], zb)
    pltpu.sync_copy(zb, degp_o.at[c, pl.ds(s * (NP // NS), NP // NS)])

    def _rw(t, _):
        buf2d.at[t // 8][pl.ds((t % 8) * 16, 16)] = rows1[pl.ds(t * 16, 16)]
        return 0
    lax.fori_loop(0, CAP // 16, _rw, 0)
    pltpu.sync_copy(buf2d, lrow_o.at[w])

    for u in range((CAP - EPT) // 16):
        cols1[pl.ds(EPT + u * 16, 16)] = jnp.full((16,), 2 * NP, jnp.int32)

    def _rm0(t, _):
        col16 = cols1[pl.ds(t * 16, 16)]
        trash = HALF + (i16 * 64 + t) % TR
        m = col16 < HALF
        buf2d.at[t // 8][pl.ds((t % 8) * 16, 16)] = jnp.where(m, col16, trash)
        return 0
    lax.fori_loop(0, CAP // 16, _rm0, 0)
    pltpu.sync_copy(buf2d, lcol_o.at[w, 0])

    def _rm1(t, _):
        col16 = cols1[pl.ds(t * 16, 16)]
        trash = HALF + (i16 * 64 + t) % TR
        m = (col16 >= HALF) & (col16 < NP)
        buf2d.at[t // 8][pl.ds((t % 8) * 16, 16)] = jnp.where(
            m, col16 - HALF, trash)
        return 0
    lax.fori_loop(0, CAP // 16, _rm1, 0)
    pltpu.sync_copy(buf2d, lcol_o.at[w, 1])


_BLK = 256


def _tc_body(x_ref, w_ref, b_ref, degp_ref,
             outs_ref, aggi_ref, hh_ref, dinvr_ref, selfcr_ref):
    i = pl.program_id(0)
    h = jnp.dot(x_ref[...], w_ref[...], preferred_element_type=jnp.float32)
    h = h + b_ref[...][None, :]
    nrm2 = jnp.sum(h * h, axis=1, keepdims=True)
    h = h * (SCALE * lax.rsqrt(jnp.maximum(nrm2, 1e-24)))
    rows = i * _BLK + lax.broadcasted_iota(jnp.int32, (_BLK, 1), 0)
    mask = (rows < N).astype(jnp.float32)
    h = h * mask
    deg = degp_ref[0, :] + degp_ref[1, :] + 1.0
    dinv = lax.rsqrt(deg) * mask[:, 0]
    selfc = BETA * dinv * dinv
    hh = ALPHA * h
    outs_ref[...] = h * dinv[:, None]
    hh_ref[...] = hh
    aggi_ref[...] = selfc[:, None] * h + hh
    dinvr_ref[...] = jnp.broadcast_to(dinv[:, None], (_BLK, 16))
    selfcr_ref[...] = jnp.broadcast_to(selfc[:, None], (_BLK, 16))


_tc_prep = pl.pallas_call(
    _tc_body,
    grid=(NP // _BLK,),
    in_specs=[
        pl.BlockSpec((_BLK, D), lambda i: (i, 0)),
        pl.BlockSpec((D, D), lambda i: (0, 0)),
        pl.BlockSpec((D,), lambda i: (0,)),
        pl.BlockSpec((NC, _BLK), lambda i: (0, i)),
    ],
    out_specs=[
        pl.BlockSpec((_BLK, D), lambda i: (i, 0)),
        pl.BlockSpec((_BLK, D), lambda i: (i, 0)),
        pl.BlockSpec((_BLK, D), lambda i: (i, 0)),
        pl.BlockSpec((_BLK, 16), lambda i: (i, 0)),
        pl.BlockSpec((_BLK, 16), lambda i: (i, 0)),
    ],
    out_shape=[
        jax.ShapeDtypeStruct((NP, D), jnp.float32),
        jax.ShapeDtypeStruct((NP, D), jnp.float32),
        jax.ShapeDtypeStruct((NP, D), jnp.float32),
        jax.ShapeDtypeStruct((NP, 16), jnp.float32),
        jax.ShapeDtypeStruct((NP, 16), jnp.float32),
    ],
)


@functools.partial(
    pl.kernel,
    out_type=(
        jax.ShapeDtypeStruct((NP, D), jnp.float32),
        jax.ShapeDtypeStruct((NP, D), jnp.float32),
        jax.ShapeDtypeStruct((NP, D), jnp.float32),
    ),
    mesh=_mesh,
    scratch_types=(
        pltpu.VMEM_SHARED((AGR, D), jnp.float32),
        pltpu.VMEM((128,), jnp.int32),
        pltpu.VMEM((128,), jnp.int32),
        pltpu.VMEM((128,), jnp.int32),
        pltpu.VMEM((128,), jnp.int32),
        pltpu.VMEM((128, D), jnp.float32),
        pltpu.VMEM((128, D), jnp.float32),
        pltpu.VMEM((64, 16), jnp.float32),
        pltpu.VMEM((64, 16), jnp.float32),
        pltpu.SemaphoreType.DMA,
        pltpu.SemaphoreType.DMA,
        pltpu.SemaphoreType.DMA,
        pltpu.SemaphoreType.DMA,
    ),
)
def _hop(outs_i, aggi_i, hh_i, dinvr_i, selfcr_i, lrow_i, lcol_i,
         out_o, outs_o, aggi_o,
         agg_sh, lrvA, lcvA, lrvB, lcvB, bA, bB, dv_v, sc_v,
         semgA, semgB, semsA, semsB):
    c = lax.axis_index("c")
    s = lax.axis_index("s")
    zeros16 = jnp.zeros((16,), jnp.float32)

    def _zb(t, _):
        bA.at[t // 8][pl.ds((t % 8) * 16, 16)] = zeros16
        return 0
    lax.fori_loop(0, (128 * D) // 16, _zb, 0)
    pltpu.sync_copy(bA, agg_sh.at[pl.ds(s * ZPT, 128)])
    pltpu.sync_copy(bA, agg_sh.at[pl.ds(s * ZPT + 128, 128)])
    pltpu.sync_copy(bA, agg_sh.at[pl.ds(s * ZPT + 256, 128)])
    plsc.subcore_barrier()

    def _fetch(q, lrv, lcv):
        w = 2 * s + q // NCH
        kk = q % NCH
        pltpu.sync_copy(lrow_i.at[w, kk], lrv)
        pltpu.sync_copy(lcol_i.at[w, c, kk], lcv)

    def _g(lrv, buf, sem):
        pltpu.async_copy(outs_i.at[lrv], buf, sem)

    def _gw(lrv, buf, sem):
        pltpu.make_async_copy(outs_i.at[lrv], buf, sem).wait()

    def _sc(buf, lcv, sem):
        pltpu.async_copy(buf, agg_sh.at[lcv], sem, add=True)

    def _scw(buf, lcv, sem):
        pltpu.make_async_copy(buf, agg_sh.at[lcv], sem).wait()

    _fetch(0, lrvA, lcvA)
    _g(lrvA, bA, semgA)
    _fetch(1, lrvB, lcvB)
    _g(lrvB, bB, semgB)

    def _pipe(k, _):
        _gw(lrvA, bA, semgA)
        _sc(bA, lcvA, semsA)
        _gw(lrvB, bB, semgB)
        _sc(bB, lcvB, semsB)
        _scw(bA, lcvA, semsA)
        _fetch(2 * k + 2, lrvA, lcvA)
        _g(lrvA, bA, semgA)
        _scw(bB, lcvB, semsB)
        _fetch(2 * k + 3, lrvB, lcvB)
        _g(lrvB, bB, semgB)
        return 0
    lax.fori_loop(0, NCH - 1, _pipe, 0)

    _gw(lrvA, bA, semgA)
    _sc(bA, lcvA, semsA)
    _gw(lrvB, bB, semgB)
    _sc(bB, lcvB, semsB)
    _scw(bA, lcvA, semsA)
    _scw(bB, lcvB, semsB)
    plsc.subcore_barrier()

    for j in range(RPT // 64):
        gbase = c * HALF + s * RPT + j * 64
        lbase = s * RPT + j * 64
        pltpu.sync_copy(agg_sh.at[pl.ds(lbase, 64)], bA.at[pl.ds(0, 64)])
        pltpu.sync_copy(aggi_i.at[pl.ds(gbase, 64)], bA.at[pl.ds(64, 64)])
        pltpu.sync_copy(hh_i.at[pl.ds(gbase, 64)], bB.at[pl.ds(0, 64)])
        pltpu.sync_copy(dinvr_i.at[pl.ds(gbase, 64)], dv_v)
        pltpu.sync_copy(selfcr_i.at[pl.ds(gbase, 64)], sc_v)

        def _row(r, _):
            dv = dv_v.at[r][pl.ds(0, 16)]
            sc_ = sc_v.at[r][pl.ds(0, 16)]
            for q in range(D // 16):
                a = bA.at[r][pl.ds(q * 16, 16)]
                ai = bA.at[64 + r][pl.ds(q * 16, 16)]
                hhv = bB.at[r][pl.ds(q * 16, 16)]
                on = (BETA * dv) * a + ai
                bB.at[64 + r][pl.ds(q * 16, 16)] = on
                bA.at[r][pl.ds(q * 16, 16)] = on * dv
                bA.at[64 + r][pl.ds(q * 16, 16)] = on * sc_ + hhv
            return 0
        lax.fori_loop(0, 64, _row, 0)

        pltpu.sync_copy(bB.at[pl.ds(64, 64)], out_o.at[pl.ds(gbase, 64)])
        pltpu.sync_copy(bA.at[pl.ds(0, 64)], outs_o.at[pl.ds(gbase, 64)])
        pltpu.sync_copy(bA.at[pl.ds(64, 64)], aggi_o.at[pl.ds(gbase, 64)])


def kernel(x, edge_index, W1, b1):
    xp = jnp.pad(x, ((0, NP - N), (0, 0)))
    degp, lrow, lcol = _prep(edge_index[0], edge_index[1])
    outs, aggi, hh, dinvr, selfcr = _tc_prep(xp, W1, b1, degp)
    out = None
    for _ in range(K_HOPS):
        out, outs, aggi = _hop(outs, aggi, hh, dinvr, selfcr, lrow, lcol)
    return out[:N]


# speedup vs baseline: 16.7469x; 1.9738x over previous
import functools

import jax
import jax.numpy as jnp
from jax import lax
from jax.experimental import pallas as pl
from jax.experimental.pallas import tpu as pltpu
from jax.experimental.pallas import tpu_sc as plsc

N = 10000
NP = 10240
D = 128
E = 320000
K_HOPS = 10
ALPHA = 0.15
SCALE = 1.8
BETA = 1.0 - ALPHA

NC, NS = 2, 16
NTILES = NC * NS
EPT = E // NTILES
CAP = 10240
NCH = CAP // 128
TR = 256
AGR = NP + TR
ZR = AGR // NS
DR = NP // NS

_mesh = plsc.VectorSubcoreMesh(core_axis_name="c", subcore_axis_name="s")


@functools.partial(
    pl.kernel,
    out_type=(
        jax.ShapeDtypeStruct((NC, NP), jnp.float32),
        jax.ShapeDtypeStruct((NTILES, NCH, 2, 128), jnp.int32),
    ),
    mesh=_mesh,
    scratch_types=(
        pltpu.VMEM_SHARED((NP,), jnp.float32),
        pltpu.VMEM((CAP,), jnp.int32),
        pltpu.VMEM((CAP,), jnp.int32),
        pltpu.VMEM((NCH, 128), jnp.int32),
        pltpu.VMEM((NCH, 128), jnp.float32),
        pltpu.VMEM((NCH, 2, 128), jnp.int32),
        pltpu.VMEM((NP // NS,), jnp.float32),
    ),
)
def _prep(erow, ecol, degp_o, lists_o,
          deg_sh, rows1, cols1, idx2, upd2, buf3d, zb):
    c = lax.axis_index("c")
    s = lax.axis_index("s")
    w = s * NC + c
    i16 = lax.iota(jnp.int32, 16)
    zeros16 = jnp.zeros((16,), jnp.float32)
    ones16 = jnp.ones((16,), jnp.float32)

    pltpu.sync_copy(erow.at[pl.ds(w * EPT, EPT)], rows1.at[pl.ds(0, EPT)])
    pltpu.sync_copy(ecol.at[pl.ds(w * EPT, EPT)], cols1.at[pl.ds(0, EPT)])

    def _zb(i, _):
        zb[pl.ds(i * 16, 16)] = zeros16
        return 0
    lax.fori_loop(0, (NP // NS) // 16, _zb, 0)
    pltpu.sync_copy(zb, deg_sh.at[pl.ds(s * (NP // NS), NP // NS)])

    for u in range((CAP - EPT) // 16):
        rows1[pl.ds(EPT + u * 16, 16)] = N + (i16 + u * 16 + w * 16) % (NP - N)
        cols1[pl.ds(EPT + u * 16, 16)] = (i16 * 8 + u * 128) % NP

    def _fill(t, _):
        idx2.at[t // 8][pl.ds((t % 8) * 16, 16)] = cols1[pl.ds(t * 16, 16)]
        upd2.at[t // 8][pl.ds((t % 8) * 16, 16)] = ones16
        return 0
    lax.fori_loop(0, EPT // 16, _fill, 0)

    def _fillz(t, _):
        idx2.at[t // 8][pl.ds((t % 8) * 16, 16)] = cols1[pl.ds(t * 16, 16)]
        upd2.at[t // 8][pl.ds((t % 8) * 16, 16)] = zeros16
        return 0
    lax.fori_loop(EPT // 16, CAP // 16, _fillz, 0)

    plsc.subcore_barrier()

    def _deg(j, _):
        pltpu.sync_copy(upd2.at[j], deg_sh.at[idx2.at[j]], add=True)
        return 0
    lax.fori_loop(0, NCH, _deg, 0)

    plsc.subcore_barrier()
    pltpu.sync_copy(deg_sh.at[pl.ds(s * (NP // NS), NP // NS)], zb)
    pltpu.sync_copy(zb, degp_o.at[c, pl.ds(s * (NP // NS), NP // NS)])

    for u in range((CAP - EPT) // 16):
        cols1[pl.ds(EPT + u * 16, 16)] = jnp.full((16,), 2 * NP, jnp.int32)

    def _rm(t, _):
        col16 = cols1[pl.ds(t * 16, 16)]
        trash = NP + (i16 * 16 + t) % TR
        m = col16 < NP
        buf3d.at[t // 8, 0][pl.ds((t % 8) * 16, 16)] = rows1[pl.ds(t * 16, 16)]
        buf3d.at[t // 8, 1][pl.ds((t % 8) * 16, 16)] = jnp.where(m, col16, trash)
        return 0
    lax.fori_loop(0, CAP // 16, _rm, 0)
    pltpu.sync_copy(buf3d, lists_o.at[w])


_BLK = 256


def _tc_body(x_ref, w_ref, b_ref, degp_ref,
             outs_ref, aggi_ref, hh_ref, dinvr_ref, selfcr_ref):
    i = pl.program_id(0)
    h = jnp.dot(x_ref[...], w_ref[...], preferred_element_type=jnp.float32)
    h = h + b_ref[...][None, :]
    nrm2 = jnp.sum(h * h, axis=1, keepdims=True)
    h = h * (SCALE * lax.rsqrt(jnp.maximum(nrm2, 1e-24)))
    rows = i * _BLK + lax.broadcasted_iota(jnp.int32, (_BLK, 1), 0)
    mask = (rows < N).astype(jnp.float32)
    h = h * mask
    deg = degp_ref[0, :] + degp_ref[1, :] + 1.0
    dinv = lax.rsqrt(deg) * mask[:, 0]
    selfc = BETA * dinv * dinv
    hh = ALPHA * h
    outs_ref[...] = h * dinv[:, None]
    hh_ref[...] = hh
    aggi_ref[...] = selfc[:, None] * h + hh
    dinvr_ref[...] = jnp.broadcast_to(dinv[:, None], (_BLK, 16))
    selfcr_ref[...] = jnp.broadcast_to(selfc[:, None], (_BLK, 16))


_tc_prep = pl.pallas_call(
    _tc_body,
    grid=(NP // _BLK,),
    in_specs=[
        pl.BlockSpec((_BLK, D), lambda i: (i, 0)),
        pl.BlockSpec((D, D), lambda i: (0, 0)),
        pl.BlockSpec((D,), lambda i: (0,)),
        pl.BlockSpec((NC, _BLK), lambda i: (0, i)),
    ],
    out_specs=[
        pl.BlockSpec((_BLK, D), lambda i: (i, 0)),
        pl.BlockSpec((_BLK, D), lambda i: (i, 0)),
        pl.BlockSpec((_BLK, D), lambda i: (i, 0)),
        pl.BlockSpec((_BLK, 16), lambda i: (i, 0)),
        pl.BlockSpec((_BLK, 16), lambda i: (i, 0)),
    ],
    out_shape=[
        jax.ShapeDtypeStruct((NP, D), jnp.float32),
        jax.ShapeDtypeStruct((NP, D), jnp.float32),
        jax.ShapeDtypeStruct((NP, D), jnp.float32),
        jax.ShapeDtypeStruct((NP, 16), jnp.float32),
        jax.ShapeDtypeStruct((NP, 16), jnp.float32),
    ],
)


@functools.partial(
    pl.kernel,
    out_type=jax.ShapeDtypeStruct((NC, NP, D), jnp.float32),
    mesh=_mesh,
    scratch_types=(
        pltpu.VMEM_SHARED((AGR, D), jnp.float32),
        pltpu.VMEM((2, 128), jnp.int32),
        pltpu.VMEM((2, 128), jnp.int32),
        pltpu.VMEM((128, D), jnp.float32),
        pltpu.VMEM((128, D), jnp.float32),
        pltpu.SemaphoreType.DMA,
        pltpu.SemaphoreType.DMA,
        pltpu.SemaphoreType.DMA,
        pltpu.SemaphoreType.DMA,
    ),
)
def _edges(outs_i, lists_i, aggp_o,
           agg_sh, li0, li1, b0, b1, sg0, sg1, ss0, ss1):
    c = lax.axis_index("c")
    s = lax.axis_index("s")
    w = s * NC + c
    zeros16 = jnp.zeros((16,), jnp.float32)

    def _zb(t, _):
        b0.at[t // 8][pl.ds((t % 8) * 16, 16)] = zeros16
        return 0
    lax.fori_loop(0, (128 * D) // 16, _zb, 0)
    for j in range(ZR // 128):
        pltpu.sync_copy(b0, agg_sh.at[pl.ds(s * ZR + j * 128, 128)])
    pltpu.sync_copy(b0.at[pl.ds(0, ZR % 128)],
                    agg_sh.at[pl.ds(s * ZR + (ZR // 128) * 128, ZR % 128)])
    plsc.subcore_barrier()

    slots = ((li0, b0, sg0, ss0), (li1, b1, sg1, ss1))

    def _fetch(q, li):
        pltpu.sync_copy(lists_i.at[w, q], li)

    def _g(li, buf, sem):
        pltpu.async_copy(outs_i.at[li.at[0]], buf, sem)

    def _step(q, slot, nxt=None):
        li, buf, sg, ss = slot
        pltpu.make_async_copy(outs_i.at[li.at[0]], buf, sg).wait()
        pltpu.async_copy(buf, agg_sh.at[li.at[1]], ss, add=True)
        pltpu.make_async_copy(buf, agg_sh.at[li.at[1]], ss).wait()
        if nxt is not None:
            _fetch(nxt, li)
            _g(li, buf, sg)

    _fetch(0, slots[0][0])
    _g(slots[0][0], slots[0][1], slots[0][2])
    _fetch(1, slots[1][0])
    _g(slots[1][0], slots[1][1], slots[1][2])

    def _pipe(k, _):
        q = 2 * k
        _step(q, slots[0], q + 2)
        _step(q + 1, slots[1], q + 3)
        return 0
    lax.fori_loop(0, (NCH - 2) // 2, _pipe, 0)

    _step(NCH - 2, slots[0])
    _step(NCH - 1, slots[1])
    plsc.subcore_barrier()

    for j in range(DR // 128):
        pltpu.sync_copy(agg_sh.at[pl.ds(s * DR + j * 128, 128)], b0)
        pltpu.sync_copy(b0, aggp_o.at[c, pl.ds(s * DR + j * 128, 128)])


def _up_body(aggp_ref, aggi_ref, hh_ref, dv_ref, sc_ref,
             out_ref, outs_ref, aggi2_ref):
    a = aggp_ref[0] + aggp_ref[1]
    dv = dv_ref[...][:, :1]
    scv = sc_ref[...][:, :1]
    on = (BETA * dv) * a + aggi_ref[...]
    out_ref[...] = on
    outs_ref[...] = on * dv
    aggi2_ref[...] = on * scv + hh_ref[...]


_tc_update = pl.pallas_call(
    _up_body,
    grid=(NP // _BLK,),
    in_specs=[
        pl.BlockSpec((NC, _BLK, D), lambda i: (0, i, 0)),
        pl.BlockSpec((_BLK, D), lambda i: (i, 0)),
        pl.BlockSpec((_BLK, D), lambda i: (i, 0)),
        pl.BlockSpec((_BLK, 16), lambda i: (i, 0)),
        pl.BlockSpec((_BLK, 16), lambda i: (i, 0)),
    ],
    out_specs=[
        pl.BlockSpec((_BLK, D), lambda i: (i, 0)),
        pl.BlockSpec((_BLK, D), lambda i: (i, 0)),
        pl.BlockSpec((_BLK, D), lambda i: (i, 0)),
    ],
    out_shape=[
        jax.ShapeDtypeStruct((NP, D), jnp.float32),
        jax.ShapeDtypeStruct((NP, D), jnp.float32),
        jax.ShapeDtypeStruct((NP, D), jnp.float32),
    ],
)


def kernel(x, edge_index, W1, b1):
    xp = jnp.pad(x, ((0, NP - N), (0, 0)))
    degp, lists = _prep(edge_index[0], edge_index[1])
    outs, aggi, hh, dinvr, selfcr = _tc_prep(xp, W1, b1, degp)
    out = None
    for _ in range(K_HOPS):
        aggp = _edges(outs, lists)
        out, outs, aggi = _tc_update(aggp, aggi, hh, dinvr, selfcr)
    return out[:N]


# speedup vs baseline: 18.6231x; 1.1120x over previous
import functools

import jax
import jax.numpy as jnp
from jax import lax
from jax.experimental import pallas as pl
from jax.experimental.pallas import tpu as pltpu
from jax.experimental.pallas import tpu_sc as plsc

N = 10000
NP = 10240
D = 128
E = 320000
K_HOPS = 10
ALPHA = 0.15
SCALE = 1.8
BETA = 1.0 - ALPHA

NC, NS = 2, 16
NTILES = NC * NS
EPT = E // NTILES
CAP = 10240
NCH = CAP // 128
AGR = 10112
ZR = AGR // NS

_mesh = plsc.VectorSubcoreMesh(core_axis_name="c", subcore_axis_name="s")


@functools.partial(
    pl.kernel,
    out_type=(
        jax.ShapeDtypeStruct((NC, NP), jnp.float32),
        jax.ShapeDtypeStruct((NTILES, NCH, 2, 128), jnp.int32),
    ),
    mesh=_mesh,
    scratch_types=(
        pltpu.VMEM_SHARED((NP,), jnp.float32),
        pltpu.VMEM((CAP,), jnp.int32),
        pltpu.VMEM((CAP,), jnp.int32),
        pltpu.VMEM((NCH, 128), jnp.int32),
        pltpu.VMEM((NCH, 128), jnp.float32),
        pltpu.VMEM((NCH, 2, 128), jnp.int32),
        pltpu.VMEM((NP // NS,), jnp.float32),
    ),
)
def _prep(erow, ecol, degp_o, lists_o,
          deg_sh, rows1, cols1, idx2, upd2, buf3d, zb):
    c = lax.axis_index("c")
    s = lax.axis_index("s")
    w = s * NC + c
    i16 = lax.iota(jnp.int32, 16)
    zeros16 = jnp.zeros((16,), jnp.float32)
    ones16 = jnp.ones((16,), jnp.float32)

    pltpu.sync_copy(erow.at[pl.ds(w * EPT, EPT)], rows1.at[pl.ds(0, EPT)])
    pltpu.sync_copy(ecol.at[pl.ds(w * EPT, EPT)], cols1.at[pl.ds(0, EPT)])

    def _zb(i, _):
        zb[pl.ds(i * 16, 16)] = zeros16
        return 0
    lax.fori_loop(0, (NP // NS) // 16, _zb, 0)
    pltpu.sync_copy(zb, deg_sh.at[pl.ds(s * (NP // NS), NP // NS)])

    for u in range((CAP - EPT) // 16):
        rows1[pl.ds(EPT + u * 16, 16)] = N + (i16 + u * 16 + w * 16) % (NP - N)
        cols1[pl.ds(EPT + u * 16, 16)] = (i16 * 8 + u * 128) % NP

    def _fill(t, _):
        idx2.at[t // 8][pl.ds((t % 8) * 16, 16)] = cols1[pl.ds(t * 16, 16)]
        upd2.at[t // 8][pl.ds((t % 8) * 16, 16)] = ones16
        return 0
    lax.fori_loop(0, EPT // 16, _fill, 0)

    def _fillz(t, _):
        idx2.at[t // 8][pl.ds((t % 8) * 16, 16)] = cols1[pl.ds(t * 16, 16)]
        upd2.at[t // 8][pl.ds((t % 8) * 16, 16)] = zeros16
        return 0
    lax.fori_loop(EPT // 16, CAP // 16, _fillz, 0)

    plsc.subcore_barrier()

    def _deg(j, _):
        pltpu.sync_copy(upd2.at[j], deg_sh.at[idx2.at[j]], add=True)
        return 0
    lax.fori_loop(0, NCH, _deg, 0)

    plsc.subcore_barrier()
    pltpu.sync_copy(deg_sh.at[pl.ds(s * (NP // NS), NP // NS)], zb)
    pltpu.sync_copy(zb, degp_o.at[c, pl.ds(s * (NP // NS), NP // NS)])

    for u in range((CAP - EPT) // 16):
        cols1[pl.ds(EPT + u * 16, 16)] = jnp.full((16,), 2 * NP, jnp.int32)

    def _rm(t, _):
        col16 = cols1[pl.ds(t * 16, 16)]
        trash = 8000 + (i16 * 16 + t) % 2048
        m = col16 < NP
        buf3d.at[t // 8, 0][pl.ds((t % 8) * 16, 16)] = rows1[pl.ds(t * 16, 16)]
        buf3d.at[t // 8, 1][pl.ds((t % 8) * 16, 16)] = jnp.where(m, col16, trash)
        return 0
    lax.fori_loop(0, CAP // 16, _rm, 0)
    pltpu.sync_copy(buf3d, lists_o.at[w])


_BLK = 256


def _tc_body(x_ref, w_ref, b_ref, degp_ref,
             outs_ref, aggi_ref, hh_ref, dinvr_ref, selfcr_ref):
    i = pl.program_id(0)
    h = jnp.dot(x_ref[...], w_ref[...], preferred_element_type=jnp.float32)
    h = h + b_ref[...][None, :]
    nrm2 = jnp.sum(h * h, axis=1, keepdims=True)
    h = h * (SCALE * lax.rsqrt(jnp.maximum(nrm2, 1e-24)))
    rows = i * _BLK + lax.broadcasted_iota(jnp.int32, (_BLK, 1), 0)
    mask = (rows < N).astype(jnp.float32)
    h = h * mask
    deg = degp_ref[0, :] + degp_ref[1, :] + 1.0
    dinv = lax.rsqrt(deg) * mask[:, 0]
    selfc = BETA * dinv * dinv
    hh = ALPHA * h
    outs_ref[...] = h * dinv[:, None]
    hh_ref[...] = hh
    aggi_ref[...] = selfc[:, None] * h + hh
    dinvr_ref[...] = jnp.broadcast_to(dinv[:, None], (_BLK, 16))
    selfcr_ref[...] = jnp.broadcast_to(selfc[:, None], (_BLK, 16))


_tc_prep = pl.pallas_call(
    _tc_body,
    grid=(NP // _BLK,),
    in_specs=[
        pl.BlockSpec((_BLK, D), lambda i: (i, 0)),
        pl.BlockSpec((D, D), lambda i: (0, 0)),
        pl.BlockSpec((D,), lambda i: (0,)),
        pl.BlockSpec((NC, _BLK), lambda i: (0, i)),
    ],
    out_specs=[
        pl.BlockSpec((_BLK, D), lambda i: (i, 0)),
        pl.BlockSpec((_BLK, D), lambda i: (i, 0)),
        pl.BlockSpec((_BLK, D), lambda i: (i, 0)),
        pl.BlockSpec((_BLK, 16), lambda i: (i, 0)),
        pl.BlockSpec((_BLK, 16), lambda i: (i, 0)),
    ],
    out_shape=[
        jax.ShapeDtypeStruct((NP, D), jnp.float32),
        jax.ShapeDtypeStruct((NP, D), jnp.float32),
        jax.ShapeDtypeStruct((NP, D), jnp.float32),
        jax.ShapeDtypeStruct((NP, 16), jnp.float32),
        jax.ShapeDtypeStruct((NP, 16), jnp.float32),
    ],
)


@functools.partial(
    pl.kernel,
    out_type=jax.ShapeDtypeStruct((NC, NP, D), jnp.float32),
    mesh=_mesh,
    scratch_types=(
        pltpu.VMEM_SHARED((AGR, D), jnp.float32),
        pltpu.VMEM((2, 128), jnp.int32),
        pltpu.VMEM((2, 128), jnp.int32),
        pltpu.VMEM((2, 128), jnp.int32),
        pltpu.VMEM((128, D), jnp.float32),
        pltpu.VMEM((128, D), jnp.float32),
        pltpu.VMEM((128, D), jnp.float32),
        pltpu.SemaphoreType.DMA,
        pltpu.SemaphoreType.DMA,
        pltpu.SemaphoreType.DMA,
        pltpu.SemaphoreType.DMA,
        pltpu.SemaphoreType.DMA,
        pltpu.SemaphoreType.DMA,
    ),
)
def _edges(outs_i, lists_i, aggp_o,
           agg_sh, li0, li1, li2, b0, b1, b2, sg0, sg1, sg2, ss0, ss1, ss2):
    c = lax.axis_index("c")
    s = lax.axis_index("s")
    w = s * NC + c
    zeros16 = jnp.zeros((16,), jnp.float32)

    def _zb(t, _):
        b0.at[t // 8][pl.ds((t % 8) * 16, 16)] = zeros16
        return 0
    lax.fori_loop(0, (128 * D) // 16, _zb, 0)
    for j in range(ZR // 128):
        pltpu.sync_copy(b0, agg_sh.at[pl.ds(s * ZR + j * 128, 128)])
    pltpu.sync_copy(b0.at[pl.ds(0, ZR % 128)],
                    agg_sh.at[pl.ds(s * ZR + (ZR // 128) * 128, ZR % 128)])
    plsc.subcore_barrier()

    slots = ((li0, b0, sg0, ss0), (li1, b1, sg1, ss1), (li2, b2, sg2, ss2))

    def _fetch(q, li):
        pltpu.sync_copy(lists_i.at[w, q], li)

    def _g(li, buf, sem):
        pltpu.async_copy(outs_i.at[li.at[0]], buf, sem)

    def _visit(cur, ref, do_scw, nxt=None):
        li, buf, sg, ss = cur
        pltpu.make_async_copy(outs_i.at[li.at[0]], buf, sg).wait()
        pltpu.async_copy(buf, agg_sh.at[li.at[1]], ss, add=True)
        rli, rbuf, rsg, rss = ref
        if do_scw:
            pltpu.make_async_copy(rbuf, agg_sh.at[rli.at[1]], rss).wait()
        if nxt is not None:
            _fetch(nxt, rli)
            _g(rli, rbuf, rsg)

    _fetch(0, slots[0][0])
    _g(slots[0][0], slots[0][1], slots[0][2])
    _fetch(1, slots[1][0])
    _g(slots[1][0], slots[1][1], slots[1][2])
    _visit(slots[0], slots[2], False, 2)
    _visit(slots[1], slots[0], True, 3)

    def _pipe(k, _):
        q = 3 * k + 2
        _visit(slots[2], slots[1], True, q + 2)
        _visit(slots[0], slots[2], True, q + 3)
        _visit(slots[1], slots[0], True, q + 4)
        return 0
    lax.fori_loop(0, (NCH - 5) // 3, _pipe, 0)

    _visit(slots[2], slots[1], True, NCH - 1)
    _visit(slots[0], slots[2], True)
    _visit(slots[1], slots[0], True)
    li, buf, _, ss = slots[1]
    pltpu.make_async_copy(buf, agg_sh.at[li.at[1]], ss).wait()
    plsc.subcore_barrier()

    for j in range(ZR // 128):
        pltpu.sync_copy(agg_sh.at[pl.ds(s * ZR + j * 128, 128)], b0)
        pltpu.sync_copy(b0, aggp_o.at[c, pl.ds(s * ZR + j * 128, 128)])
    pltpu.sync_copy(agg_sh.at[pl.ds(s * ZR + (ZR // 128) * 128, ZR % 128)],
                    b0.at[pl.ds(0, ZR % 128)])
    pltpu.sync_copy(b0.at[pl.ds(0, ZR % 128)],
                    aggp_o.at[c, pl.ds(s * ZR + (ZR // 128) * 128, ZR % 128)])


def _up_body(aggp_ref, aggi_ref, hh_ref, dv_ref, sc_ref,
             out_ref, outs_ref, aggi2_ref):
    i = pl.program_id(0)
    rows = i * _BLK + lax.broadcasted_iota(jnp.int32, (_BLK, 1), 0)
    a = jnp.where(rows < N, aggp_ref[0] + aggp_ref[1], 0.0)
    dv = dv_ref[...][:, :1]
    scv = sc_ref[...][:, :1]
    on = (BETA * dv) * a + aggi_ref[...]
    out_ref[...] = on
    outs_ref[...] = on * dv
    aggi2_ref[...] = on * scv + hh_ref[...]


_tc_update = pl.pallas_call(
    _up_body,
    grid=(NP // _BLK,),
    in_specs=[
        pl.BlockSpec((NC, _BLK, D), lambda i: (0, i, 0)),
        pl.BlockSpec((_BLK, D), lambda i: (i, 0)),
        pl.BlockSpec((_BLK, D), lambda i: (i, 0)),
        pl.BlockSpec((_BLK, 16), lambda i: (i, 0)),
        pl.BlockSpec((_BLK, 16), lambda i: (i, 0)),
    ],
    out_specs=[
        pl.BlockSpec((_BLK, D), lambda i: (i, 0)),
        pl.BlockSpec((_BLK, D), lambda i: (i, 0)),
        pl.BlockSpec((_BLK, D), lambda i: (i, 0)),
    ],
    out_shape=[
        jax.ShapeDtypeStruct((NP, D), jnp.float32),
        jax.ShapeDtypeStruct((NP, D), jnp.float32),
        jax.ShapeDtypeStruct((NP, D), jnp.float32),
    ],
)


def kernel(x, edge_index, W1, b1):
    xp = jnp.pad(x, ((0, NP - N), (0, 0)))
    degp, lists = _prep(edge_index[0], edge_index[1])
    outs, aggi, hh, dinvr, selfcr = _tc_prep(xp, W1, b1, degp)
    out = None
    for _ in range(K_HOPS):
        aggp = _edges(outs, lists)
        out, outs, aggi = _tc_update(aggp, aggi, hh, dinvr, selfcr)
    return out[:N]
